# Initial kernel scaffold; baseline (speedup 1.0000x reference)
import jax
import jax.numpy as jnp
from jax.experimental import pallas as pl


def kernel(x, labels, centers):
    raise NotImplementedError("write your pallas kernel here")


# speedup vs baseline: 2.6340x; 2.6340x over previous
import jax
import jax.numpy as jnp
from jax import lax
from jax.experimental import pallas as pl
from jax.experimental.pallas import tpu as pltpu
from jax.experimental.pallas import tpu_sc as plsc

NUM_CLASSES = 100000
FEAT_DIM = 64
BATCH = 16384
ALPHA = 0.5

_NTILES = 16
_PER_TILE = BATCH // _NTILES
_IDXW = 128
_NIDX = _PER_TILE // _IDXW
_CNTW = 16


def _sc_body(x_hbm, lbl_hbm, ids_hbm, part_out,
             tr_sh, s_sh, c_sh,
             lbl_v, ids_v, r_v, xbuf, mbuf, cbuf, pbuf, ones_v):
    wid = lax.axis_index("s")
    row0 = wid * _NIDX
    samp0 = wid * _PER_TILE

    zeros16 = jnp.zeros((16,), jnp.float32)
    ones16 = jnp.ones((16,), jnp.float32)

    def _zx(i, _):
        xbuf[i, pl.ds(0, 16)] = zeros16
        xbuf[i, pl.ds(16, 16)] = zeros16
        xbuf[i, pl.ds(32, 16)] = zeros16
        xbuf[i, pl.ds(48, 16)] = zeros16
        pbuf[i, pl.ds(0, _CNTW)] = zeros16
        ones_v[i, pl.ds(0, _CNTW)] = ones16
        return 0
    lax.fori_loop(0, _IDXW, _zx, 0)

    for j in range(_NIDX):
        pltpu.sync_copy(xbuf, s_sh.at[pl.ds(samp0 + j * _IDXW, _IDXW)])
        pltpu.sync_copy(pbuf, c_sh.at[pl.ds(samp0 + j * _IDXW, _IDXW)])

    pltpu.sync_copy(lbl_hbm.at[pl.ds(row0, _NIDX)], lbl_v)
    pltpu.sync_copy(ids_hbm.at[pl.ds(row0, _NIDX)], ids_v)
    for j in range(_NIDX):
        pltpu.sync_copy(ids_v.at[j], tr_sh.at[lbl_v.at[j]])

    plsc.subcore_barrier()

    for j in range(_NIDX):
        pltpu.sync_copy(tr_sh.at[lbl_v.at[j]], r_v.at[j])
        pltpu.sync_copy(x_hbm.at[pl.ds(samp0 + j * _IDXW, _IDXW)], xbuf)
        pltpu.sync_copy(xbuf, s_sh.at[r_v.at[j]], add=True)
        pltpu.sync_copy(ones_v, c_sh.at[r_v.at[j]], add=True)

    plsc.subcore_barrier()

    for j in range(_NIDX):
        pltpu.sync_copy(x_hbm.at[pl.ds(samp0 + j * _IDXW, _IDXW)], xbuf)
        pltpu.sync_copy(s_sh.at[r_v.at[j]], mbuf)
        pltpu.sync_copy(c_sh.at[r_v.at[j]], cbuf)

        def _dist(i, _):
            inv = ones16 / cbuf[i, pl.ds(0, _CNTW)]
            d0 = xbuf[i, pl.ds(0, 16)] - mbuf[i, pl.ds(0, 16)] * inv
            d1 = xbuf[i, pl.ds(16, 16)] - mbuf[i, pl.ds(16, 16)] * inv
            d2 = xbuf[i, pl.ds(32, 16)] - mbuf[i, pl.ds(32, 16)] * inv
            d3 = xbuf[i, pl.ds(48, 16)] - mbuf[i, pl.ds(48, 16)] * inv
            pbuf[i, pl.ds(0, _CNTW)] = d0 * d0 + d1 * d1 + d2 * d2 + d3 * d3
            return 0
        lax.fori_loop(0, _IDXW, _dist, 0)
        pltpu.sync_copy(pbuf, part_out.at[pl.ds(samp0 + j * _IDXW, _IDXW)])


def _make_sc_call():
    mesh = plsc.VectorSubcoreMesh(core_axis_name="c", subcore_axis_name="s",
                                  num_cores=1)
    return pl.kernel(
        _sc_body,
        out_type=jax.ShapeDtypeStruct((BATCH, _CNTW), jnp.float32),
        mesh=mesh,
        scratch_types=[
            pltpu.VMEM_SHARED((NUM_CLASSES,), jnp.int32),
            pltpu.VMEM_SHARED((BATCH, FEAT_DIM), jnp.float32),
            pltpu.VMEM_SHARED((BATCH, _CNTW), jnp.float32),
            pltpu.VMEM((_NIDX, _IDXW), jnp.int32),
            pltpu.VMEM((_NIDX, _IDXW), jnp.int32),
            pltpu.VMEM((_NIDX, _IDXW), jnp.int32),
            pltpu.VMEM((_IDXW, FEAT_DIM), jnp.float32),
            pltpu.VMEM((_IDXW, FEAT_DIM), jnp.float32),
            pltpu.VMEM((_IDXW, _CNTW), jnp.float32),
            pltpu.VMEM((_IDXW, _CNTW), jnp.float32),
            pltpu.VMEM((_IDXW, _CNTW), jnp.float32),
        ],
        compiler_params=pltpu.CompilerParams(use_tc_tiling_on_sc=False),
        name="center_loss_sc",
    )


def _finish_body(p_ref, o_ref):
    ssq = jnp.sum(p_ref[...], axis=1, keepdims=True)
    o_ref[0, 0] = jnp.sum(jnp.sqrt(ssq)) * (ALPHA / BATCH)


_finish = pl.pallas_call(
    _finish_body,
    out_shape=jax.ShapeDtypeStruct((1, 1), jnp.float32),
    out_specs=pl.BlockSpec(memory_space=pltpu.SMEM),
    name="center_loss_tc_finish",
)


@jax.jit
def kernel(x, labels, centers):
    del centers
    lbl2d = labels.astype(jnp.int32).reshape(BATCH // _IDXW, _IDXW)
    ids2d = jnp.arange(BATCH, dtype=jnp.int32).reshape(BATCH // _IDXW, _IDXW)
    partials = _make_sc_call()(x, lbl2d, ids2d)
    loss = _finish(partials)
    return loss[0, 0]
